# baseline (speedup 1.0000x reference)
import jax
import jax.numpy as jnp
from jax.experimental import pallas as pl


def _body(idx_ref, feat_ref, tab_ref, out_ref):
    T = idx_ref.shape[1]
    V = tab_ref.shape[1]
    bB = idx_ref.shape[0]
    ids = idx_ref[...]
    embeds = []
    for t in range(T):
        onehot = (
            jax.lax.broadcasted_iota(jnp.int32, (bB, V), 1) == ids[:, t : t + 1]
        ).astype(jnp.float32)
        e = jax.lax.dot_general(
            onehot,
            tab_ref[t],
            (((1,), (0,)), ((), ())),
            preferred_element_type=jnp.float32,
        )
        embeds.append(e[:, None, :])
    out_ref[:, 0:T, :] = jnp.concatenate(embeds, axis=1)
    out_ref[:, T:, :] = feat_ref[...]


def kernel(feature, indices, tables):
    B, L, D = feature.shape
    T, V, _ = tables.shape
    idx = indices[..., 0].astype(jnp.int32).T

    bB = 64
    grid = (B // bB,)

    out = pl.pallas_call(
        _body,
        grid=grid,
        in_specs=[
            pl.BlockSpec((bB, T), lambda i: (i, 0)),
            pl.BlockSpec((bB, L, D), lambda i: (i, 0, 0)),
            pl.BlockSpec((T, V, D), lambda i: (0, 0, 0)),
        ],
        out_specs=pl.BlockSpec((bB, L + T, D), lambda i: (i, 0, 0)),
        out_shape=jax.ShapeDtypeStruct((B, L + T, D), jnp.float32),
    )(idx, feature, tables)
    return out
